# Initial kernel scaffold; baseline (speedup 1.0000x reference)
import jax
import jax.numpy as jnp
from jax.experimental import pallas as pl


def kernel(hidden_states, router_w, w1, w2, bias):
    raise NotImplementedError("write your pallas kernel here")


# speedup vs baseline: 5.0351x; 5.0351x over previous
import functools

import jax
import jax.numpy as jnp
from jax import lax
from jax.experimental import pallas as pl
from jax.experimental.pallas import tpu as pltpu

TOP_K = 2
_SQRT_HALF = 0.7071067811865476


def _gelu_exact(x):
    return 0.5 * x * (1.0 + lax.erf(x * _SQRT_HALF))


def _dense_body(x_ref, rw_ref, w1_ref, w2_ref, bias_ref, out_ref, c_ref):
    e = pl.program_id(0)
    i = pl.program_id(1)
    T, E = c_ref.shape

    @pl.when(jnp.logical_and(e == 0, i == 0))
    def _():
        logits = lax.dot_general(
            x_ref[...], rw_ref[...],
            (((1,), (1,)), ((), ())),
            preferred_element_type=jnp.float32,
        )
        m = jnp.max(logits, axis=-1, keepdims=True)
        ex = jnp.exp(logits - m)
        p = ex / jnp.sum(ex, axis=-1, keepdims=True)
        eidx = lax.broadcasted_iota(jnp.int32, (T, E), 1)
        big = jnp.int32(E + 1)
        m1 = jnp.max(p, axis=-1, keepdims=True)
        a1 = jnp.min(jnp.where(p == m1, eidx, big), axis=-1, keepdims=True)
        oh1 = eidx == a1
        p2 = jnp.where(oh1, -jnp.inf, p)
        m2 = jnp.max(p2, axis=-1, keepdims=True)
        a2 = jnp.min(jnp.where(p2 == m2, eidx, big), axis=-1, keepdims=True)
        oh2 = eidx == a2
        c_ref[...] = jnp.where(oh1 | oh2, p, 0.0)
        out_ref[...] = jnp.broadcast_to(bias_ref[...][None, :], out_ref.shape)

    h = lax.dot_general(
        x_ref[...], w1_ref[0],
        (((1,), (1,)), ((), ())),
        preferred_element_type=jnp.float32,
    )
    a = _gelu_exact(h)
    part = lax.dot_general(
        a, w2_ref[0],
        (((1,), (1,)), ((), ())),
        preferred_element_type=jnp.float32,
    )
    c_all = c_ref[...]
    eidx = lax.broadcasted_iota(jnp.int32, c_all.shape, 1)
    c_col = jnp.sum(jnp.where(eidx == e, c_all, 0.0), axis=-1, keepdims=True)
    out_ref[...] += part * c_col


def kernel(hidden_states, router_w, w1, w2, bias):
    T, H = hidden_states.shape
    E, I, _ = w1.shape
    IT = min(512, I)
    grid = (E, I // IT)
    return pl.pallas_call(
        _dense_body,
        grid=grid,
        in_specs=[
            pl.BlockSpec((T, H), lambda e, i: (0, 0)),
            pl.BlockSpec((E, H), lambda e, i: (0, 0)),
            pl.BlockSpec((1, IT, H), lambda e, i: (e, i, 0)),
            pl.BlockSpec((1, H, IT), lambda e, i: (e, 0, i)),
            pl.BlockSpec((H,), lambda e, i: (0,)),
        ],
        out_specs=pl.BlockSpec((T, H), lambda e, i: (0, 0)),
        out_shape=jax.ShapeDtypeStruct((T, H), jnp.float32),
        scratch_shapes=[pltpu.VMEM((T, E), jnp.float32)],
    )(hidden_states, router_w, w1, w2, bias)
